# Initial kernel scaffold; baseline (speedup 1.0000x reference)
import jax
import jax.numpy as jnp
from jax.experimental import pallas as pl


def kernel(node_features, direction_matrix, port_feature_mask):
    raise NotImplementedError("write your pallas kernel here")


# speedup vs baseline: 3090.5657x; 3090.5657x over previous
import functools

import jax
import jax.numpy as jnp
from jax.experimental import pallas as pl


def _pack_kernel(nf_ref, w_ref, m_ref, r_ref):
    w = w_ref[...]
    b_dim = nf_ref.shape[0]
    d_idx = jax.lax.broadcasted_iota(jnp.int32, (1, 6), 1)
    wm = 1 << d_idx
    wr = 1 << ((d_idx + 3) % 6)
    for b in range(b_dim):
        act = jnp.dot(nf_ref[b], w, preferred_element_type=jnp.float32)
        pht = (act > 0).astype(jnp.int32)
        m_ref[b, :] = jnp.sum(pht * wm, axis=1)
        r_ref[b, :] = jnp.sum(pht * wr, axis=1)


def _main_kernel(dir_ref, m_ref, r_ref, out_ref):
    d = dir_ref[...]
    b_dim = out_ref.shape[0]
    for b in range(b_dim):
        mb = m_ref[b, :][:, None]
        rb = r_ref[b, :][None, :]
        combined = mb & rb
        out_ref[b] = ((combined >> d) & 1).astype(jnp.float32)


@functools.partial(jax.jit, static_argnames=())
def kernel(node_features, direction_matrix, port_feature_mask):
    B, N, F = node_features.shape
    dir32 = direction_matrix.astype(jnp.int32)

    m, r = pl.pallas_call(
        _pack_kernel,
        out_shape=(
            jax.ShapeDtypeStruct((B, N), jnp.int32),
            jax.ShapeDtypeStruct((B, N), jnp.int32),
        ),
    )(node_features, port_feature_mask)

    BI, BJ = 256, 1024
    grid = (N // BI, N // BJ)
    out = pl.pallas_call(
        _main_kernel,
        grid=grid,
        in_specs=[
            pl.BlockSpec((BI, BJ), lambda i, j: (i, j)),
            pl.BlockSpec((B, BI), lambda i, j: (0, i)),
            pl.BlockSpec((B, BJ), lambda i, j: (0, j)),
        ],
        out_specs=pl.BlockSpec((B, BI, BJ), lambda i, j: (0, i, j)),
        out_shape=jax.ShapeDtypeStruct((B, N, N), jnp.float32),
    )(dir32, m, r)
    return out


# speedup vs baseline: 3275.1795x; 1.0597x over previous
import functools

import jax
import jax.numpy as jnp
from jax.experimental import pallas as pl


def _pack_kernel(nf_ref, w_ref, m_ref, r_ref):
    w = w_ref[...]
    b_dim = nf_ref.shape[0]
    d_idx = jax.lax.broadcasted_iota(jnp.int32, (1, 6), 1)
    wm = 1 << d_idx
    wr = 1 << ((d_idx + 3) % 6)
    for b in range(b_dim):
        act = jnp.dot(nf_ref[b], w, preferred_element_type=jnp.float32)
        pht = (act > 0).astype(jnp.int32)
        m_ref[b, :] = jnp.sum(pht * wm, axis=1)
        r_ref[b, :] = jnp.sum(pht * wr, axis=1)


def _main_kernel(dir_ref, m_ref, r_ref, out_ref):
    d = dir_ref[...]
    b_dim = out_ref.shape[0]
    for b in range(b_dim):
        mb = m_ref[b, :][:, None]
        rb = r_ref[b, :][None, :]
        combined = mb & rb
        out_ref[b] = ((combined >> d) & 1).astype(jnp.float32)


@functools.partial(jax.jit, static_argnames=())
def kernel(node_features, direction_matrix, port_feature_mask):
    B, N, F = node_features.shape
    dir32 = direction_matrix.astype(jnp.int32)

    m, r = pl.pallas_call(
        _pack_kernel,
        out_shape=(
            jax.ShapeDtypeStruct((B, N), jnp.int32),
            jax.ShapeDtypeStruct((B, N), jnp.int32),
        ),
    )(node_features, port_feature_mask)

    BI, BJ = 256, 2048
    grid = (N // BI, N // BJ)
    out = pl.pallas_call(
        _main_kernel,
        grid=grid,
        in_specs=[
            pl.BlockSpec((BI, BJ), lambda i, j: (i, j)),
            pl.BlockSpec((B, BI), lambda i, j: (0, i)),
            pl.BlockSpec((B, BJ), lambda i, j: (0, j)),
        ],
        out_specs=pl.BlockSpec((B, BI, BJ), lambda i, j: (0, i, j)),
        out_shape=jax.ShapeDtypeStruct((B, N, N), jnp.float32),
    )(dir32, m, r)
    return out


# speedup vs baseline: 3313.9468x; 1.0118x over previous
import functools

import jax
import jax.numpy as jnp
from jax.experimental import pallas as pl


def _pack_kernel(nf_ref, w_ref, m_ref, r_ref):
    w = w_ref[...]
    b_dim = nf_ref.shape[0]
    d_idx = jax.lax.broadcasted_iota(jnp.int32, (1, 6), 1)
    wm = 1 << d_idx
    wr = 1 << ((d_idx + 3) % 6)
    for b in range(b_dim):
        act = jnp.dot(nf_ref[b], w, preferred_element_type=jnp.float32)
        pht = (act > 0).astype(jnp.int32)
        m_ref[b, :] = jnp.sum(pht * wm, axis=1)
        r_ref[b, :] = jnp.sum(pht * wr, axis=1)


def _main_kernel(dir_ref, m_ref, r_ref, out_ref):
    d = dir_ref[...]
    b_dim = out_ref.shape[0]
    for b in range(b_dim):
        mb = m_ref[b, :][:, None]
        rb = r_ref[b, :][None, :]
        combined = mb & rb
        out_ref[b] = ((combined >> d) & 1).astype(jnp.float32)


@functools.partial(jax.jit, static_argnames=())
def kernel(node_features, direction_matrix, port_feature_mask):
    B, N, F = node_features.shape
    dir32 = direction_matrix.astype(jnp.int32)

    m, r = pl.pallas_call(
        _pack_kernel,
        out_shape=(
            jax.ShapeDtypeStruct((B, N), jnp.int32),
            jax.ShapeDtypeStruct((B, N), jnp.int32),
        ),
    )(node_features, port_feature_mask)

    BI, BJ = 512, 2048
    grid = (N // BI, N // BJ)
    out = pl.pallas_call(
        _main_kernel,
        grid=grid,
        in_specs=[
            pl.BlockSpec((BI, BJ), lambda i, j: (i, j)),
            pl.BlockSpec((B, BI), lambda i, j: (0, i)),
            pl.BlockSpec((B, BJ), lambda i, j: (0, j)),
        ],
        out_specs=pl.BlockSpec((B, BI, BJ), lambda i, j: (0, i, j)),
        out_shape=jax.ShapeDtypeStruct((B, N, N), jnp.float32),
    )(dir32, m, r)
    return out
